# baseline (speedup 1.0000x reference)
import jax
import jax.numpy as jnp
from jax import lax
from jax.experimental import pallas as pl
from jax.experimental.pallas import tpu as pltpu
from jax.experimental.pallas import tpu_sc as plsc

B, C, N0, N1, K = 4, 128, 40962, 10242, 6
BC = B * C
NC, NS = 2, 16
NW = NC * NS
N1P = 10752
VPW = N1P // NW
BLK = 16
NB = VPW // BLK
GR = BLK * K
DV = BC // 16


def _sc_body(xt_hbm, ni_hbm, out_hbm, ibuf, gbuf, obuf,
             gsem0, gsem1, osem0, osem1):
    wid = lax.axis_index("s") * NC + lax.axis_index("c")
    v0 = wid * VPW
    gsems = (gsem0, gsem1)
    osems = (osem0, osem1)

    pltpu.sync_copy(ni_hbm.at[pl.ds(v0 * K, VPW * K)], ibuf)

    NSTR = 4
    SR = GR // NSTR

    def gather_start(g, ph):
        for q in range(NSTR):
            pltpu.make_async_copy(
                xt_hbm.at[ibuf.at[pl.ds(g * GR + q * SR, SR)]],
                gbuf.at[ph, pl.ds(q * SR, SR)], gsems[ph]).start()

    def gather_wait(g, ph):
        pltpu.make_async_copy(xt_hbm.at[ibuf.at[pl.ds(g * GR, GR)]],
                              gbuf.at[ph], gsems[ph]).wait()

    def out_wait(ph):
        pltpu.make_async_copy(obuf.at[ph], out_hbm.at[pl.ds(v0, BLK)],
                              osems[ph]).wait()

    def compute(g, ph):
        @plsc.parallel_loop(0, BLK, 1)
        def body(v):
            r0 = v * K
            for bb in range(2):
                for dc in range(C // 16):
                    d = dc * 16
                    gl = [plsc.bitcast(gbuf[ph, r0 + j, bb, pl.ds(d, 16)],
                                       jnp.bfloat16) for j in range(K)]
                    m = jnp.maximum(
                        jnp.maximum(jnp.maximum(gl[0], gl[1]),
                                    jnp.maximum(gl[2], gl[3])),
                        jnp.maximum(gl[4], gl[5]))
                    obuf[ph, v, bb, pl.ds(d, 16)] = plsc.bitcast(m, jnp.int32)

        pltpu.make_async_copy(
            obuf.at[ph], out_hbm.at[pl.ds(v0 + g * BLK, BLK)],
            osems[ph]).start()

    gather_start(0, 0)

    def pair(p, _):
        for ph in range(2):
            g = 2 * p + ph
            nxt = 1 - ph

            @pl.when(g + 1 < NB)
            def _():
                gather_start(g + 1, nxt)

            gather_wait(g, ph)

            @pl.when(g >= 2)
            def _():
                out_wait(ph)

            compute(g, ph)
        return 0

    lax.fori_loop(0, NB // 2, pair, 0)
    if NB % 2:
        g = NB - 1
        gather_wait(g, 0)
        out_wait(0)
        compute(g, 0)
    out_wait(0)
    out_wait(1)


def kernel(input, in_ni):
    xb = input.transpose(2, 0, 1).astype(jnp.bfloat16)
    xt = jax.lax.bitcast_convert_type(
        xb.reshape(N0, 2, C, 2), jnp.int32)
    ni = in_ni.astype(jnp.int32)
    ni = jnp.pad(ni, ((0, N1P - N1), (0, 0))).reshape(-1)
    mesh = plsc.VectorSubcoreMesh(core_axis_name="c", subcore_axis_name="s")
    out = pl.kernel(
        _sc_body,
        mesh=mesh,
        compiler_params=pltpu.CompilerParams(needs_layout_passes=False),
        out_type=jax.ShapeDtypeStruct((N1P, 2, C), jnp.int32),
        scratch_types=[
            pltpu.VMEM((VPW * K,), jnp.int32),
            pltpu.VMEM((2, GR, 2, C), jnp.int32),
            pltpu.VMEM((2, BLK, 2, C), jnp.int32),
            pltpu.SemaphoreType.DMA,
            pltpu.SemaphoreType.DMA,
            pltpu.SemaphoreType.DMA,
            pltpu.SemaphoreType.DMA,
        ],
    )(xt, ni)
    ob = jax.lax.bitcast_convert_type(out, jnp.bfloat16).reshape(N1P, BC)
    return ob[:N1].T.reshape(B, C, N1).astype(jnp.float32)


# speedup vs baseline: 4.0241x; 4.0241x over previous
import jax
import jax.numpy as jnp
from jax import lax
from jax.experimental import pallas as pl
from jax.experimental.pallas import tpu as pltpu
from jax.experimental.pallas import tpu_sc as plsc

B, C, N0, N1, K = 4, 128, 40962, 10242, 6
BC = B * C
NC, NS = 2, 16
NW = NC * NS
N1P = 10752
VPW = N1P // NW
BLK = 16
NB = VPW // BLK
GR = BLK * K
DV = BC // 16


def _sc_body(xt_hbm, ni_hbm, out_hbm, ibuf, sbuf, ismem, gbuf, obuf,
             gsem0, gsem1, osem0, osem1):
    wid = lax.axis_index("s") * NC + lax.axis_index("c")
    sid = lax.axis_index("s")
    v0 = wid * VPW
    gsems = (gsem0, gsem1)
    osems = (osem0, osem1)

    pltpu.sync_copy(ni_hbm.at[pl.ds(v0 * K, VPW * K)], ibuf)
    pltpu.sync_copy(ibuf, sbuf.at[pl.ds(sid * VPW * K, VPW * K)])

    def gather_start(g, ph):
        pltpu.sync_copy(sbuf.at[pl.ds(sid * VPW * K + g * GR, GR)],
                        ismem.at[ph])

        def row(r, _):
            idx = ismem[ph, r]
            pltpu.make_async_copy(xt_hbm.at[pl.ds(idx, 1)],
                                  gbuf.at[ph, pl.ds(r, 1)],
                                  gsems[ph]).start()
            return 0

        lax.fori_loop(0, GR, row, 0)

    def gather_wait(g, ph):
        pltpu.make_async_copy(xt_hbm.at[pl.ds(0, GR)],
                              gbuf.at[ph], gsems[ph]).wait()

    def out_wait(ph):
        pltpu.make_async_copy(obuf.at[ph], out_hbm.at[pl.ds(v0, BLK)],
                              osems[ph]).wait()

    def compute(g, ph):
        @plsc.parallel_loop(0, BLK, 1)
        def body(v):
            r0 = v * K
            for bb in range(B):
                for dc in range(C // 16):
                    d = dc * 16
                    gl = [gbuf[ph, r0 + j, bb, pl.ds(d, 16)]
                          for j in range(K)]
                    m = jnp.maximum(
                        jnp.maximum(jnp.maximum(gl[0], gl[1]),
                                    jnp.maximum(gl[2], gl[3])),
                        jnp.maximum(gl[4], gl[5]))
                    obuf[ph, v, bb, pl.ds(d, 16)] = m

        pltpu.make_async_copy(
            obuf.at[ph], out_hbm.at[pl.ds(v0 + g * BLK, BLK)],
            osems[ph]).start()

    gather_start(0, 0)

    def pair(p, _):
        for ph in range(2):
            g = 2 * p + ph
            nxt = 1 - ph

            @pl.when(g + 1 < NB)
            def _():
                gather_start(g + 1, nxt)

            gather_wait(g, ph)

            @pl.when(g >= 2)
            def _():
                out_wait(ph)

            compute(g, ph)
        return 0

    lax.fori_loop(0, NB // 2, pair, 0)
    if NB % 2:
        g = NB - 1
        gather_wait(g, 0)
        out_wait(0)
        compute(g, 0)
    out_wait(0)
    out_wait(1)


def kernel(input, in_ni):
    xt = input.transpose(2, 0, 1)
    ni = in_ni.astype(jnp.int32)
    ni = jnp.pad(ni, ((0, N1P - N1), (0, 0))).reshape(-1)
    mesh = plsc.VectorSubcoreMesh(core_axis_name="c", subcore_axis_name="s")
    out = pl.kernel(
        _sc_body,
        mesh=mesh,
        out_type=jax.ShapeDtypeStruct((N1P, B, C), jnp.float32),
        scratch_types=[
            pltpu.VMEM((VPW * K,), jnp.int32),
            pltpu.VMEM_SHARED((NS * VPW * K,), jnp.int32),
            pltpu.SMEM((2, GR), jnp.int32),
            pltpu.VMEM((2, GR, B, C), jnp.float32),
            pltpu.VMEM((2, BLK, B, C), jnp.float32),
            pltpu.SemaphoreType.DMA,
            pltpu.SemaphoreType.DMA,
            pltpu.SemaphoreType.DMA,
            pltpu.SemaphoreType.DMA,
        ],
    )(xt, ni)
    return out[:N1].transpose(1, 2, 0)


# speedup vs baseline: 4.1667x; 1.0354x over previous
import jax
import jax.numpy as jnp
from jax import lax
from jax.experimental import pallas as pl
from jax.experimental.pallas import tpu as pltpu
from jax.experimental.pallas import tpu_sc as plsc

B, C, N0, N1, K = 4, 128, 40962, 10242, 6
BC = B * C
NC, NS = 2, 16
NW = NC * NS
N1P = 10752
VPW = N1P // NW
BLK = 16
NB = VPW // BLK
GR = BLK * K
DV = BC // 16


def _sc_body(xt_hbm, ni_hbm, out_hbm, ibuf, gbuf, obuf,
             gsem0, gsem1, osem0, osem1):
    wid = lax.axis_index("s") * NC + lax.axis_index("c")
    v0 = wid * VPW
    gsems = (gsem0, gsem1)
    osems = (osem0, osem1)

    pltpu.sync_copy(ni_hbm.at[pl.ds(v0 * K, VPW * K)], ibuf)

    NSTR = 4
    SR = GR // NSTR

    def gather_start(g, ph):
        for q in range(NSTR):
            pltpu.make_async_copy(
                xt_hbm.at[ibuf.at[pl.ds(g * GR + q * SR, SR)]],
                gbuf.at[ph, pl.ds(q * SR, SR)], gsems[ph]).start()

    def gather_wait(g, ph):
        pltpu.make_async_copy(xt_hbm.at[ibuf.at[pl.ds(g * GR, GR)]],
                              gbuf.at[ph], gsems[ph]).wait()

    def out_wait(ph):
        pltpu.make_async_copy(obuf.at[ph], out_hbm.at[pl.ds(v0, BLK)],
                              osems[ph]).wait()

    def compute(g, ph):
        @plsc.parallel_loop(0, BLK, 1)
        def body(v):
            r0 = v * K
            for bb in range(B):
                for dc in range(C // 16):
                    d = dc * 16
                    gl = [gbuf[ph, r0 + j, bb, pl.ds(d, 16)]
                          for j in range(K)]
                    m = jnp.maximum(
                        jnp.maximum(jnp.maximum(gl[0], gl[1]),
                                    jnp.maximum(gl[2], gl[3])),
                        jnp.maximum(gl[4], gl[5]))
                    obuf[ph, v, bb, pl.ds(d, 16)] = m

        pltpu.make_async_copy(
            obuf.at[ph], out_hbm.at[pl.ds(v0 + g * BLK, BLK)],
            osems[ph]).start()

    gather_start(0, 0)

    def pair(p, _):
        for ph in range(2):
            g = 2 * p + ph
            nxt = 1 - ph

            @pl.when(g + 1 < NB)
            def _():
                gather_start(g + 1, nxt)

            gather_wait(g, ph)

            @pl.when(g >= 2)
            def _():
                out_wait(ph)

            compute(g, ph)
        return 0

    lax.fori_loop(0, NB // 2, pair, 0)
    if NB % 2:
        g = NB - 1
        gather_wait(g, 0)
        out_wait(0)
        compute(g, 0)
    out_wait(0)
    out_wait(1)


def kernel(input, in_ni):
    xt = input.transpose(2, 0, 1)
    ni = in_ni.astype(jnp.int32).reshape(-1)
    ni = jnp.pad(ni, (0, (N1P - N1) * K))
    mesh = plsc.VectorSubcoreMesh(core_axis_name="c", subcore_axis_name="s")
    out = pl.kernel(
        _sc_body,
        mesh=mesh,
        out_type=jax.ShapeDtypeStruct((N1P, B, C), jnp.float32),
        scratch_types=[
            pltpu.VMEM((VPW * K,), jnp.int32),
            pltpu.VMEM((2, GR, B, C), jnp.float32),
            pltpu.VMEM((2, BLK, B, C), jnp.float32),
            pltpu.SemaphoreType.DMA,
            pltpu.SemaphoreType.DMA,
            pltpu.SemaphoreType.DMA,
            pltpu.SemaphoreType.DMA,
        ],
    )(xt, ni)
    return out[:N1].transpose(1, 2, 0)
